# baseline (speedup 1.0000x reference)
import jax
import jax.numpy as jnp
from jax.experimental import pallas as pl
from jax.experimental.pallas import tpu as pltpu

CH = 256
NBUF = 16


def _pe_add_manual(x_ref, pe_ref, o_ref, x_buf, out_buf, pe_buf,
                   x_sems, pe_sems, out_sems):
    B = x_ref.shape[0]
    n_j = x_ref.shape[1] // CH
    n_chunks = n_j * B

    def chunk_bj(c):
        return c % B, c // B

    def x_copy(c):
        b, j = chunk_bj(c)
        slot = c % NBUF
        return pltpu.make_async_copy(
            x_ref.at[b, pl.ds(j * CH, CH), :], x_buf.at[slot], x_sems.at[slot])

    def pe_copy(j):
        return pltpu.make_async_copy(
            pe_ref.at[pl.ds(j * CH, CH), :], pe_buf.at[j], pe_sems.at[j])

    def out_copy(c):
        b, j = chunk_bj(c)
        slot = c % NBUF
        return pltpu.make_async_copy(
            out_buf.at[slot], o_ref.at[b, pl.ds(j * CH, CH), :],
            out_sems.at[slot])

    x_copy(0).start()
    pe_copy(0).start()
    for c in range(1, NBUF):
        x_copy(c).start()
    for j in range(1, n_j):
        pe_copy(j).start()

    for c in range(n_chunks):
        b, j = chunk_bj(c)
        slot = c % NBUF
        x_copy(c).wait()
        if b == 0:
            pe_copy(j).wait()
        if c >= NBUF:
            out_copy(c - NBUF).wait()
        out_buf[slot] = x_buf[slot] + pe_buf[j]
        out_copy(c).start()
        if c + NBUF < n_chunks:
            x_copy(c + NBUF).start()

    for c in range(n_chunks - NBUF, n_chunks):
        out_copy(c).wait()


def kernel(x, pos_emb):
    B, S, D = x.shape
    n_j = S // CH
    return pl.pallas_call(
        _pe_add_manual,
        in_specs=[
            pl.BlockSpec(memory_space=pl.ANY),
            pl.BlockSpec(memory_space=pl.ANY),
        ],
        out_specs=pl.BlockSpec(memory_space=pl.ANY),
        out_shape=jax.ShapeDtypeStruct(x.shape, x.dtype),
        scratch_shapes=[
            pltpu.VMEM((NBUF, CH, D), jnp.float32),
            pltpu.VMEM((NBUF, CH, D), jnp.float32),
            pltpu.VMEM((n_j, CH, D), jnp.float32),
            pltpu.SemaphoreType.DMA((NBUF,)),
            pltpu.SemaphoreType.DMA((n_j,)),
            pltpu.SemaphoreType.DMA((NBUF,)),
        ],
    )(x, pos_emb)


# speedup vs baseline: 1.0151x; 1.0151x over previous
import jax
import jax.numpy as jnp
from jax.experimental import pallas as pl
from jax.experimental.pallas import tpu as pltpu

CH = 1024
NBUF = 5


def _pe_add_manual(x_ref, pe_ref, o_ref, x_buf, out_buf, pe_buf,
                   x_sems, pe_sems, out_sems):
    B = x_ref.shape[0]
    n_j = x_ref.shape[1] // CH
    n_chunks = n_j * B

    def chunk_bj(c):
        return c % B, c // B

    def x_copy(c):
        b, j = chunk_bj(c)
        slot = c % NBUF
        return pltpu.make_async_copy(
            x_ref.at[b, pl.ds(j * CH, CH), :], x_buf.at[slot], x_sems.at[slot])

    def pe_copy(j):
        return pltpu.make_async_copy(
            pe_ref.at[pl.ds(j * CH, CH), :], pe_buf.at[j], pe_sems.at[j])

    def out_copy(c):
        b, j = chunk_bj(c)
        slot = c % NBUF
        return pltpu.make_async_copy(
            out_buf.at[slot], o_ref.at[b, pl.ds(j * CH, CH), :],
            out_sems.at[slot])

    x_copy(0).start()
    pe_copy(0).start()
    for c in range(1, NBUF):
        x_copy(c).start()
    for j in range(1, n_j):
        pe_copy(j).start()

    for c in range(n_chunks):
        b, j = chunk_bj(c)
        slot = c % NBUF
        x_copy(c).wait()
        if b == 0:
            pe_copy(j).wait()
        if c >= NBUF:
            out_copy(c - NBUF).wait()
        out_buf[slot] = x_buf[slot] + pe_buf[j]
        out_copy(c).start()
        if c + NBUF < n_chunks:
            x_copy(c + NBUF).start()

    for c in range(n_chunks - NBUF, n_chunks):
        out_copy(c).wait()


def kernel(x, pos_emb):
    B, S, D = x.shape
    n_j = S // CH
    return pl.pallas_call(
        _pe_add_manual,
        in_specs=[
            pl.BlockSpec(memory_space=pl.ANY),
            pl.BlockSpec(memory_space=pl.ANY),
        ],
        out_specs=pl.BlockSpec(memory_space=pl.ANY),
        out_shape=jax.ShapeDtypeStruct(x.shape, x.dtype),
        scratch_shapes=[
            pltpu.VMEM((NBUF, CH, D), jnp.float32),
            pltpu.VMEM((NBUF, CH, D), jnp.float32),
            pltpu.VMEM((n_j, CH, D), jnp.float32),
            pltpu.SemaphoreType.DMA((NBUF,)),
            pltpu.SemaphoreType.DMA((n_j,)),
            pltpu.SemaphoreType.DMA((NBUF,)),
        ],
    )(x, pos_emb)
